# baseline (speedup 1.0000x reference)
import jax
import jax.numpy as jnp
from jax.experimental import pallas as pl
import jax.experimental.pallas.tpu as pltpu
import jax.experimental.pallas.tpu_sc as plsc

N_K = 1024
D = 256
B = 16384
BT = 1024
GW = 128


def _argmin_body(z_ref, emb_ref, idx_ref):
    z = z_ref[...]
    emb = emb_ref[...]
    zsq = jnp.sum(z * z, axis=-1, keepdims=True)
    esq = jnp.sum(emb * emb, axis=-1)
    s = jax.lax.dot_general(
        z.astype(jnp.bfloat16), emb.astype(jnp.bfloat16),
        (((1,), (1,)), ((), ())),
        preferred_element_type=jnp.float32)
    d = (zsq + esq[None, :]) - 2.0 * s
    m = jnp.min(d, axis=-1, keepdims=True)
    iota = jax.lax.broadcasted_iota(jnp.int32, d.shape, 1)
    idx_ref[...] = jnp.min(jnp.where(d == m, iota, N_K), axis=-1)


def _tc_argmin(z, emb):
    return pl.pallas_call(
        _argmin_body,
        grid=(B // BT,),
        in_specs=[
            pl.BlockSpec((BT, D), lambda i: (i, 0)),
            pl.BlockSpec((N_K, D), lambda i: (0, 0)),
        ],
        out_specs=pl.BlockSpec((BT,), lambda i: (i,)),
        out_shape=jax.ShapeDtypeStruct((B,), jnp.int32),
    )(z, emb)


def _sc_gather(emb, idx):
    idx2 = idx.reshape((1, B))

    @pl.kernel(
        out_type=jax.ShapeDtypeStruct((B, D), jnp.float32),
        mesh=plsc.VectorSubcoreMesh(
            core_axis_name="core", subcore_axis_name="subcore"),
    )
    def gather_kernel(emb_hbm, i_hbm, o_hbm):
        def body(i_vmem, o_vmem):
            pltpu.sync_copy(emb_hbm.at[i_vmem.at[0]], o_vmem)

        pltpu.emit_pipeline(
            body,
            grid=(B // GW,),
            in_specs=[pl.BlockSpec((1, GW), index_map=lambda i: (0, i))],
            out_specs=[pl.BlockSpec((GW, D), index_map=lambda i: (i, 0))],
            core_axis_name=("core", "subcore"),
            dimension_semantics=(pltpu.PARALLEL,),
        )(i_hbm, o_hbm)

    return gather_kernel(emb, idx2)


def kernel(z, emb):
    idx = _tc_argmin(z, emb)
    zq = _sc_gather(emb, idx)
    return (zq, idx)
